# baseline (speedup 1.0000x reference)
import functools
import math

import jax
import jax.numpy as jnp
import numpy as np
from jax import lax
from jax.experimental import pallas as pl
from jax.experimental.pallas import tpu as pltpu
from jax.experimental.pallas import tpu_sc as plsc

VOCAB = 100000
D_MODEL = 768
MAX_LEN = 4096
BATCH = 4
SCALE = math.sqrt(D_MODEL)

N_ROWS = MAX_LEN * BATCH
NW = 32
ROWS_PER_W = N_ROWS // NW
CHUNK_ROWS = 32
N_CHUNKS = ROWS_PER_W // CHUNK_ROWS
L_PER_CHUNK = CHUNK_ROWS // BATCH
LANES = 16
C_PER_ROW = D_MODEL // LANES


def _pe_table():
    pe = np.zeros((MAX_LEN, D_MODEL), dtype=np.float32)
    position = np.arange(0, MAX_LEN, dtype=np.float32)[:, None]
    div_term = np.exp(
        np.arange(0, D_MODEL, 2, dtype=np.float32) * (-math.log(10000.0) / D_MODEL)
    )
    pe[:, 0::2] = np.sin(position * div_term)
    pe[:, 1::2] = np.cos(position * div_term)
    return jnp.asarray(pe)


_MESH = plsc.VectorSubcoreMesh(core_axis_name="c", subcore_axis_name="s")


@functools.partial(
    pl.kernel,
    mesh=_MESH,
    out_type=jax.ShapeDtypeStruct((N_ROWS, D_MODEL), jnp.float32),
    scratch_types=[
        pltpu.VMEM((N_CHUNKS, CHUNK_ROWS), jnp.int32),
        pltpu.VMEM((2, CHUNK_ROWS, D_MODEL), jnp.float32),
        pltpu.VMEM((2, CHUNK_ROWS, D_MODEL), jnp.float32),
        pltpu.VMEM((2, L_PER_CHUNK, D_MODEL), jnp.float32),
        pltpu.SemaphoreType.DMA,
        pltpu.SemaphoreType.DMA,
        pltpu.SemaphoreType.DMA,
        pltpu.SemaphoreType.DMA,
        pltpu.SemaphoreType.DMA,
        pltpu.SemaphoreType.DMA,
    ],
)
def _embed_sc(table_hbm, idx_hbm, pe_hbm, out_hbm,
              idx_v, in_v, out_v, pe_v,
              gsem0, gsem1, psem0, psem1, osem0, osem1):
    wid = lax.axis_index("s") * 2 + lax.axis_index("c")
    base_row = wid * ROWS_PER_W
    base_l = wid * (ROWS_PER_W // BATCH)
    gsems = (gsem0, gsem1)
    psems = (psem0, psem1)
    osems = (osem0, osem1)

    pltpu.sync_copy(idx_hbm.at[pl.ds(wid * N_CHUNKS, N_CHUNKS)], idx_v)

    def start_in(g, s):
        l0 = base_l + g * L_PER_CHUNK
        pltpu.async_copy(pe_hbm.at[pl.ds(l0, L_PER_CHUNK)], pe_v.at[s], psems[s])
        pltpu.async_copy(table_hbm.at[idx_v.at[g]], in_v.at[s], gsems[s])

    def wait_in(s):
        pltpu.make_async_copy(
            pe_hbm.at[pl.ds(0, L_PER_CHUNK)], pe_v.at[s], psems[s]).wait()
        pltpu.make_async_copy(
            table_hbm.at[pl.ds(0, CHUNK_ROWS)], in_v.at[s], gsems[s]).wait()

    def wait_out(s):
        pltpu.make_async_copy(
            out_hbm.at[pl.ds(0, CHUNK_ROWS)], out_v.at[s], osems[s]).wait()

    def do_chunk(g, s):
        @pl.when(g + 1 < N_CHUNKS)
        def _():
            start_in(g + 1, 1 - s)

        wait_in(s)

        @pl.when(g >= 2)
        def _():
            wait_out(s)

        def l_body(li, c2):
            for c in range(C_PER_ROW):
                pe_c = pe_v[s, li, pl.ds(c * LANES, LANES)]
                for b in range(BATCH):
                    r = li * BATCH + b
                    out_v[s, r, pl.ds(c * LANES, LANES)] = (
                        in_v[s, r, pl.ds(c * LANES, LANES)] * SCALE + pe_c
                    )
            return c2

        lax.fori_loop(0, L_PER_CHUNK, l_body, 0)
        r0 = base_row + g * CHUNK_ROWS
        pltpu.async_copy(out_v.at[s], out_hbm.at[pl.ds(r0, CHUNK_ROWS)], osems[s])

    start_in(0, 0)

    def pair_body(i, carry):
        do_chunk(2 * i, 0)
        do_chunk(2 * i + 1, 1)
        return carry

    lax.fori_loop(0, N_CHUNKS // 2, pair_body, 0)
    wait_out(0)
    wait_out(1)


def kernel(encoded_words, embed_weight):
    idx2d = encoded_words.reshape(NW * N_CHUNKS, CHUNK_ROWS)
    pe = _pe_table()
    out = _embed_sc(embed_weight, idx2d, pe)
    return out.reshape(MAX_LEN, BATCH, D_MODEL)


# speedup vs baseline: 1.1710x; 1.1710x over previous
import functools
import math

import jax
import jax.numpy as jnp
import numpy as np
from jax import lax
from jax.experimental import pallas as pl
from jax.experimental.pallas import tpu as pltpu
from jax.experimental.pallas import tpu_sc as plsc

VOCAB = 100000
D_MODEL = 768
MAX_LEN = 4096
BATCH = 4
SCALE = math.sqrt(D_MODEL)

N_ROWS = MAX_LEN * BATCH
NW = 32
ROWS_PER_W = N_ROWS // NW
CHUNK_ROWS = 32
N_CHUNKS = ROWS_PER_W // CHUNK_ROWS
L_PER_CHUNK = CHUNK_ROWS // BATCH
LANES = 16
C_PER_ROW = D_MODEL // LANES


def _pe_table():
    pe = np.zeros((MAX_LEN, D_MODEL), dtype=np.float32)
    position = np.arange(0, MAX_LEN, dtype=np.float32)[:, None]
    div_term = np.exp(
        np.arange(0, D_MODEL, 2, dtype=np.float32) * (-math.log(10000.0) / D_MODEL)
    )
    pe[:, 0::2] = np.sin(position * div_term)
    pe[:, 1::2] = np.cos(position * div_term)
    return jnp.asarray(pe)


_MESH = plsc.VectorSubcoreMesh(core_axis_name="c", subcore_axis_name="s")


@functools.partial(
    pl.kernel,
    mesh=_MESH,
    out_type=jax.ShapeDtypeStruct((N_ROWS, D_MODEL), jnp.float32),
    scratch_types=[
        pltpu.VMEM((N_CHUNKS, CHUNK_ROWS), jnp.int32),
        pltpu.VMEM((2, CHUNK_ROWS, D_MODEL), jnp.float32),
        pltpu.VMEM((2, CHUNK_ROWS, D_MODEL), jnp.float32),
        pltpu.VMEM((2, L_PER_CHUNK, D_MODEL), jnp.float32),
        pltpu.SemaphoreType.DMA,
        pltpu.SemaphoreType.DMA,
        pltpu.SemaphoreType.DMA,
        pltpu.SemaphoreType.DMA,
        pltpu.SemaphoreType.DMA,
        pltpu.SemaphoreType.DMA,
    ],
)
def _embed_sc(table_hbm, idx_hbm, pe_hbm, out_hbm,
              idx_v, in_v, out_v, pe_v,
              gsem0, gsem1, psem0, psem1, osem0, osem1):
    wid = lax.axis_index("s") * 2 + lax.axis_index("c")
    base_row = wid * ROWS_PER_W
    base_l = wid * (ROWS_PER_W // BATCH)
    gsems = (gsem0, gsem1)
    psems = (psem0, psem1)
    osems = (osem0, osem1)

    pltpu.sync_copy(idx_hbm.at[pl.ds(wid * N_CHUNKS, N_CHUNKS)], idx_v)

    def start_in(g, s):
        l0 = base_l + g * L_PER_CHUNK
        pltpu.async_copy(pe_hbm.at[pl.ds(l0, L_PER_CHUNK)], pe_v.at[s], psems[s])
        pltpu.async_copy(table_hbm.at[idx_v.at[g]], in_v.at[s], gsems[s])

    def wait_in(s):
        pltpu.make_async_copy(
            pe_hbm.at[pl.ds(0, L_PER_CHUNK)], pe_v.at[s], psems[s]).wait()
        pltpu.make_async_copy(
            table_hbm.at[pl.ds(0, CHUNK_ROWS)], in_v.at[s], gsems[s]).wait()

    def wait_out(s):
        pltpu.make_async_copy(
            out_hbm.at[pl.ds(0, CHUNK_ROWS)], out_v.at[s], osems[s]).wait()

    def do_chunk(g, s):
        @pl.when(g + 1 < N_CHUNKS)
        def _():
            start_in(g + 1, 1 - s)

        wait_in(s)

        @pl.when(g >= 2)
        def _():
            wait_out(s)

        @plsc.parallel_loop(0, L_PER_CHUNK)
        def _(li):
            for c in range(C_PER_ROW):
                pe_c = pe_v[s, li, pl.ds(c * LANES, LANES)]
                for b in range(BATCH):
                    r = li * BATCH + b
                    out_v[s, r, pl.ds(c * LANES, LANES)] = (
                        in_v[s, r, pl.ds(c * LANES, LANES)] * SCALE + pe_c
                    )
        r0 = base_row + g * CHUNK_ROWS
        pltpu.async_copy(out_v.at[s], out_hbm.at[pl.ds(r0, CHUNK_ROWS)], osems[s])

    start_in(0, 0)

    def pair_body(i, carry):
        do_chunk(2 * i, 0)
        do_chunk(2 * i + 1, 1)
        return carry

    lax.fori_loop(0, N_CHUNKS // 2, pair_body, 0)
    wait_out(0)
    wait_out(1)


def kernel(encoded_words, embed_weight):
    idx2d = encoded_words.reshape(NW * N_CHUNKS, CHUNK_ROWS)
    pe = _pe_table()
    out = _embed_sc(embed_weight, idx2d, pe)
    return out.reshape(MAX_LEN, BATCH, D_MODEL)


# speedup vs baseline: 1.5658x; 1.3372x over previous
import functools
import math

import jax
import jax.numpy as jnp
import numpy as np
from jax import lax
from jax.experimental import pallas as pl
from jax.experimental.pallas import tpu as pltpu
from jax.experimental.pallas import tpu_sc as plsc

VOCAB = 100000
D_MODEL = 768
MAX_LEN = 4096
BATCH = 4
SCALE = math.sqrt(D_MODEL)

N_ROWS = MAX_LEN * BATCH
NW = 32
ROWS_PER_W = N_ROWS // NW
CHUNK_ROWS = 32
N_CHUNKS = ROWS_PER_W // CHUNK_ROWS
L_PER_CHUNK = CHUNK_ROWS // BATCH
LANES = 16
C_PER_ROW = D_MODEL // LANES


def _pe_table():
    pe = np.zeros((MAX_LEN, D_MODEL), dtype=np.float32)
    position = np.arange(0, MAX_LEN, dtype=np.float32)[:, None]
    div_term = np.exp(
        np.arange(0, D_MODEL, 2, dtype=np.float32) * (-math.log(10000.0) / D_MODEL)
    )
    pe[:, 0::2] = np.sin(position * div_term)
    pe[:, 1::2] = np.cos(position * div_term)
    return jnp.asarray(pe)


_MESH = plsc.VectorSubcoreMesh(core_axis_name="c", subcore_axis_name="s")


@functools.partial(
    pl.kernel,
    mesh=_MESH,
    out_type=jax.ShapeDtypeStruct((MAX_LEN, BATCH, D_MODEL), jnp.float32),
    scratch_types=[
        pltpu.VMEM((N_CHUNKS, CHUNK_ROWS), jnp.int32),
        pltpu.VMEM((2, CHUNK_ROWS, D_MODEL), jnp.float32),
        pltpu.VMEM((2, L_PER_CHUNK, BATCH, D_MODEL), jnp.float32),
        pltpu.VMEM((2, L_PER_CHUNK, D_MODEL), jnp.float32),
        pltpu.SemaphoreType.DMA,
        pltpu.SemaphoreType.DMA,
        pltpu.SemaphoreType.DMA,
        pltpu.SemaphoreType.DMA,
        pltpu.SemaphoreType.DMA,
        pltpu.SemaphoreType.DMA,
    ],
)
def _embed_sc(table_hbm, idx_hbm, pe_hbm, out_hbm,
              idx_v, in_v, out_v, pe_v,
              gsem0, gsem1, psem0, psem1, osem0, osem1):
    wid = lax.axis_index("s") * 2 + lax.axis_index("c")
    base_row = wid * ROWS_PER_W
    base_l = wid * (ROWS_PER_W // BATCH)
    gsems = (gsem0, gsem1)
    psems = (psem0, psem1)
    osems = (osem0, osem1)

    pltpu.sync_copy(idx_hbm.at[pl.ds(wid * N_CHUNKS, N_CHUNKS)], idx_v)

    def start_in(g, s):
        l0 = base_l + g * L_PER_CHUNK
        pltpu.async_copy(pe_hbm.at[pl.ds(l0, L_PER_CHUNK)], pe_v.at[s], psems[s])
        pltpu.async_copy(table_hbm.at[idx_v.at[g]], in_v.at[s], gsems[s])

    def wait_in(s):
        pltpu.make_async_copy(
            pe_hbm.at[pl.ds(0, L_PER_CHUNK)], pe_v.at[s], psems[s]).wait()
        pltpu.make_async_copy(
            table_hbm.at[pl.ds(0, CHUNK_ROWS)], in_v.at[s], gsems[s]).wait()

    def wait_out(s):
        pltpu.make_async_copy(
            out_hbm.at[pl.ds(0, L_PER_CHUNK)], out_v.at[s], osems[s]).wait()

    def do_chunk(g, s):
        @pl.when(g + 1 < N_CHUNKS)
        def _():
            start_in(g + 1, 1 - s)

        wait_in(s)

        @pl.when(g >= 2)
        def _():
            wait_out(s)

        @plsc.parallel_loop(0, L_PER_CHUNK)
        def _(li):
            for c in range(C_PER_ROW):
                pe_c = pe_v[s, li, pl.ds(c * LANES, LANES)]
                for b in range(BATCH):
                    r = li * BATCH + b
                    out_v[s, li, b, pl.ds(c * LANES, LANES)] = (
                        in_v[s, r, pl.ds(c * LANES, LANES)] * SCALE + pe_c
                    )
        l0 = base_l + g * L_PER_CHUNK
        pltpu.async_copy(out_v.at[s], out_hbm.at[pl.ds(l0, L_PER_CHUNK)], osems[s])

    start_in(0, 0)

    def pair_body(i, carry):
        do_chunk(2 * i, 0)
        do_chunk(2 * i + 1, 1)
        return carry

    lax.fori_loop(0, N_CHUNKS // 2, pair_body, 0)
    wait_out(0)
    wait_out(1)


def kernel(encoded_words, embed_weight):
    idx2d = encoded_words.reshape(NW * N_CHUNKS, CHUNK_ROWS)
    pe = _pe_table()
    return _embed_sc(embed_weight, idx2d, pe)


# speedup vs baseline: 3.1702x; 2.0246x over previous
import functools
import math

import jax
import jax.numpy as jnp
import numpy as np
from jax import lax
from jax.experimental import pallas as pl
from jax.experimental.pallas import tpu as pltpu
from jax.experimental.pallas import tpu_sc as plsc

VOCAB = 100000
D_MODEL = 768
MAX_LEN = 4096
BATCH = 4
SCALE = math.sqrt(D_MODEL)

N_ROWS = MAX_LEN * BATCH
NW = 32
ROWS_PER_W = N_ROWS // NW
CHUNK_ROWS = 32
N_CHUNKS = ROWS_PER_W // CHUNK_ROWS
L_PER_CHUNK = CHUNK_ROWS // BATCH
LANES = 16
C_PER_ROW = D_MODEL // LANES


def _pe_table():
    pe = np.zeros((MAX_LEN, D_MODEL), dtype=np.float32)
    position = np.arange(0, MAX_LEN, dtype=np.float32)[:, None]
    div_term = np.exp(
        np.arange(0, D_MODEL, 2, dtype=np.float32) * (-math.log(10000.0) / D_MODEL)
    )
    pe[:, 0::2] = np.sin(position * div_term)
    pe[:, 1::2] = np.cos(position * div_term)
    return jnp.asarray(pe)


_MESH = plsc.VectorSubcoreMesh(core_axis_name="c", subcore_axis_name="s")


@functools.partial(
    pl.kernel,
    mesh=_MESH,
    out_type=jax.ShapeDtypeStruct((MAX_LEN, BATCH, D_MODEL), jnp.float32),
    scratch_types=[
        pltpu.VMEM((N_CHUNKS, CHUNK_ROWS), jnp.int32),
        pltpu.VMEM((2, CHUNK_ROWS, D_MODEL), jnp.float32),
        pltpu.VMEM((2, L_PER_CHUNK, BATCH, D_MODEL), jnp.float32),
        pltpu.VMEM((2, L_PER_CHUNK, D_MODEL), jnp.float32),
        pltpu.SemaphoreType.DMA,
        pltpu.SemaphoreType.DMA,
        pltpu.SemaphoreType.DMA,
        pltpu.SemaphoreType.DMA,
        pltpu.SemaphoreType.DMA,
        pltpu.SemaphoreType.DMA,
    ],
)
def _embed_sc(table_hbm, idx_hbm, pe_hbm, out_hbm,
              idx_v, in_v, out_v, pe_v,
              gsem0, gsem1, psem0, psem1, osem0, osem1):
    wid = lax.axis_index("s") * 2 + lax.axis_index("c")
    base_row = wid * ROWS_PER_W
    base_l = wid * (ROWS_PER_W // BATCH)
    gsems = (gsem0, gsem1)
    psems = (psem0, psem1)
    osems = (osem0, osem1)

    pltpu.sync_copy(idx_hbm.at[pl.ds(wid * N_CHUNKS, N_CHUNKS)], idx_v)

    def start_in(g, s):
        l0 = base_l + g * L_PER_CHUNK
        pltpu.async_copy(pe_hbm.at[pl.ds(l0, L_PER_CHUNK)], pe_v.at[s], psems[s])
        pltpu.async_copy(table_hbm.at[idx_v.at[g]], in_v.at[s], gsems[s])

    def wait_in(s):
        pltpu.make_async_copy(
            pe_hbm.at[pl.ds(0, L_PER_CHUNK)], pe_v.at[s], psems[s]).wait()
        pltpu.make_async_copy(
            table_hbm.at[pl.ds(0, CHUNK_ROWS)], in_v.at[s], gsems[s]).wait()

    def wait_out(s):
        pltpu.make_async_copy(
            out_hbm.at[pl.ds(0, L_PER_CHUNK)], out_v.at[s], osems[s]).wait()

    def do_chunk(g, s):
        @pl.when(g + 1 < N_CHUNKS)
        def _():
            start_in(g + 1, 1 - s)

        wait_in(s)

        @pl.when(g >= 2)
        def _():
            wait_out(s)

        @plsc.parallel_loop(0, L_PER_CHUNK)
        def _(li):
            base_r = li * BATCH

            def load_group(c):
                sl = pl.ds(c * LANES, LANES)
                return (pe_v[s, li, sl],
                        [in_v[s, base_r + b, sl] for b in range(BATCH)])

            grp = {c: load_group(c) for c in range(2)}
            for c in range(C_PER_ROW):
                pe_c, ins = grp.pop(c)
                if c + 2 < C_PER_ROW:
                    grp[c + 2] = load_group(c + 2)
                sl = pl.ds(c * LANES, LANES)
                for b in range(BATCH):
                    out_v[s, li, b, sl] = ins[b] * SCALE + pe_c
        l0 = base_l + g * L_PER_CHUNK
        pltpu.async_copy(out_v.at[s], out_hbm.at[pl.ds(l0, L_PER_CHUNK)], osems[s])

    start_in(0, 0)

    def pair_body(i, carry):
        do_chunk(2 * i, 0)
        do_chunk(2 * i + 1, 1)
        return carry

    lax.fori_loop(0, N_CHUNKS // 2, pair_body, 0)
    wait_out(0)
    wait_out(1)


def kernel(encoded_words, embed_weight):
    idx2d = encoded_words.reshape(NW * N_CHUNKS, CHUNK_ROWS)
    pe = _pe_table()
    return _embed_sc(embed_weight, idx2d, pe)
